# baseline (speedup 1.0000x reference)
import functools

import jax
import jax.numpy as jnp
from jax import lax
from jax.experimental import pallas as pl
from jax.experimental.pallas import tpu as pltpu
from jax.experimental.pallas import tpu_sc as plsc

BATCH = 16384
DIM = 64
L = 16
SUB = 8

_info = plsc.get_sparse_core_info()
NC, NS = _info.num_cores, _info.num_subcores
NW = NC * NS
BPW = BATCH // NW
CH = 128
NCH = BPW // CH
WAVE = 32

_mesh = plsc.VectorSubcoreMesh(core_axis_name="c", subcore_axis_name="s")


_GDN = lax.GatherDimensionNumbers(
    offset_dims=(), collapsed_slice_dims=(0,), start_index_map=(0,))


def _permute(v, idx):
    return lax.gather(v, idx[:, None], dimension_numbers=_GDN,
                      slice_sizes=(1,),
                      mode=lax.GatherScatterMode.PROMISE_IN_BOUNDS)


def _fold16(vecs, lane):
    def rot(v, d):
        return _permute(v, (lane + d) & (L - 1))

    dist = L // 2
    while len(vecs) > 1:
        mask = (lane & (2 * dist - 1)) < dist
        nxt = []
        for i in range(0, len(vecs), 2):
            a2 = vecs[i] + rot(vecs[i], dist)
            b2 = vecs[i + 1] + rot(vecs[i + 1], L - dist)
            nxt.append(jnp.where(mask, a2, b2))
        vecs = nxt
        dist //= 2
    bitrev = (((lane & 1) << 3) | ((lane & 2) << 1)
              | ((lane & 4) >> 1) | ((lane & 8) >> 3))
    return _permute(vecs[0], bitrev)


@functools.partial(
    pl.kernel,
    mesh=_mesh,
    out_type=(
        jax.ShapeDtypeStruct((BATCH,), jnp.float32),
        jax.ShapeDtypeStruct((BATCH,), jnp.float32),
    ),
    scratch_types=[
        pltpu.VMEM((BPW,), jnp.int32),
        pltpu.VMEM((BPW,), jnp.int32),
        pltpu.VMEM((BPW,), jnp.int32),
        pltpu.VMEM((WAVE, SUB, DIM), jnp.float32),
        pltpu.VMEM((WAVE, SUB, DIM), jnp.float32),
        pltpu.VMEM((WAVE, SUB, DIM), jnp.float32),
        pltpu.VMEM((BPW,), jnp.float32),
        pltpu.VMEM((BPW,), jnp.float32),
        pltpu.SemaphoreType.DMA,
    ],
)
def _bpr_sc(bu, bp, bn, ue, ie, pos_out, neg_out,
            idx_u, idx_p, idx_n, u_slab, p_slab, n_slab,
            pos_s, neg_s, sem):
    wid = lax.axis_index("s") * NC + lax.axis_index("c")
    base = wid * BPW

    descs = []
    for src, dst in ((bu, idx_u), (bp, idx_p), (bn, idx_n)):
        for j in range(NCH):
            descs.append(
                pltpu.async_copy(src.at[pl.ds(base + j * CH, CH)],
                                 dst.at[pl.ds(j * CH, CH)], sem))
    for d in descs:
        d.wait()

    lane = lax.iota(jnp.int32, L)

    def wave(w, carry):
        k0 = pl.multiple_of(w * WAVE, WAVE)
        vu = idx_u[pl.ds(k0, WAVE)]
        vp = idx_p[pl.ds(k0, WAVE)]
        vn = idx_n[pl.ds(k0, WAVE)]
        descs = []
        subs = []
        for i in range(WAVE):
            trip = []
            for vec, tab, slab in ((vu, ue, u_slab), (vp, ie, p_slab),
                                   (vn, ie, n_slab)):
                r = vec[i]
                r8 = pl.multiple_of((r >> 3) << 3, SUB)
                trip.append(r & 7)
                descs.append(
                    pltpu.async_copy(tab.at[pl.ds(r8, SUB), :],
                                     slab.at[i], sem))
            subs.append(trip)
        for d in descs:
            d.wait()

        for g in range(WAVE // L):
            pp, pn = [], []
            for li in range(L):
                i = g * L + li
                su, sp, sn = subs[i]
                acc_p = acc_n = None
                for c in range(DIM // L):
                    u = u_slab[i, su, pl.ds(c * L, L)]
                    tp = u * p_slab[i, sp, pl.ds(c * L, L)]
                    tn = u * n_slab[i, sn, pl.ds(c * L, L)]
                    acc_p = tp if acc_p is None else acc_p + tp
                    acc_n = tn if acc_n is None else acc_n + tn
                pp.append(acc_p)
                pn.append(acc_n)
            pos_s[pl.ds(k0 + g * L, L)] = _fold16(pp, lane)
            neg_s[pl.ds(k0 + g * L, L)] = _fold16(pn, lane)
        return carry

    lax.fori_loop(0, BPW // WAVE, wave, 0)

    pltpu.sync_copy(pos_s, pos_out.at[pl.ds(base, BPW)])
    pltpu.sync_copy(neg_s, neg_out.at[pl.ds(base, BPW)])


def _tr_body(x_ref, o_ref):
    x = x_ref[...]
    eye = (lax.broadcasted_iota(jnp.int32, (DIM, DIM), 0)
           == lax.broadcasted_iota(jnp.int32, (DIM, DIM), 1)
           ).astype(jnp.float32)
    o_ref[...] = lax.dot_general(x, eye, (((0,), (0,)), ((), ())),
                                 preferred_element_type=jnp.float32)


def _transpose_table(tt):
    n = tt.shape[1]
    blk = 4096
    return pl.pallas_call(
        _tr_body,
        grid=(pl.cdiv(n, blk),),
        in_specs=[pl.BlockSpec((DIM, blk), lambda j: (0, j))],
        out_specs=pl.BlockSpec((blk, DIM), lambda j: (j, 0)),
        out_shape=jax.ShapeDtypeStruct((n, DIM), jnp.float32),
    )(tt)


def kernel(batch_user, batch_pos_item, batch_neg_item, user_emb, item_emb):
    ue = _transpose_table(user_emb.T)
    ie = _transpose_table(item_emb.T)
    pos, neg = _bpr_sc(batch_user, batch_pos_item, batch_neg_item, ue, ie)
    return pos.reshape(BATCH, 1), neg.reshape(BATCH, 1)


# speedup vs baseline: 1.0229x; 1.0229x over previous
import functools

import jax
import jax.numpy as jnp
from jax import lax
from jax.experimental import pallas as pl
from jax.experimental.pallas import tpu as pltpu
from jax.experimental.pallas import tpu_sc as plsc

BATCH = 16384
DIM = 64
L = 16
SUB = 8

_info = plsc.get_sparse_core_info()
NC, NS = _info.num_cores, _info.num_subcores
NW = NC * NS
BPW = BATCH // NW
CH = 128
NCH = BPW // CH
WAVE = 32

_mesh = plsc.VectorSubcoreMesh(core_axis_name="c", subcore_axis_name="s")


_GDN = lax.GatherDimensionNumbers(
    offset_dims=(), collapsed_slice_dims=(0,), start_index_map=(0,))


def _permute(v, idx):
    return lax.gather(v, idx[:, None], dimension_numbers=_GDN,
                      slice_sizes=(1,),
                      mode=lax.GatherScatterMode.PROMISE_IN_BOUNDS)


def _fold16(vecs, lane):
    def rot(v, d):
        return _permute(v, (lane + d) & (L - 1))

    dist = L // 2
    while len(vecs) > 1:
        mask = (lane & (2 * dist - 1)) < dist
        nxt = []
        for i in range(0, len(vecs), 2):
            a2 = vecs[i] + rot(vecs[i], dist)
            b2 = vecs[i + 1] + rot(vecs[i + 1], L - dist)
            nxt.append(jnp.where(mask, a2, b2))
        vecs = nxt
        dist //= 2
    bitrev = (((lane & 1) << 3) | ((lane & 2) << 1)
              | ((lane & 4) >> 1) | ((lane & 8) >> 3))
    return _permute(vecs[0], bitrev)


@functools.partial(
    pl.kernel,
    mesh=_mesh,
    out_type=(
        jax.ShapeDtypeStruct((BATCH,), jnp.float32),
        jax.ShapeDtypeStruct((BATCH,), jnp.float32),
    ),
    scratch_types=[
        pltpu.VMEM((BPW,), jnp.int32),
        pltpu.VMEM((BPW,), jnp.int32),
        pltpu.VMEM((BPW,), jnp.int32),
        pltpu.VMEM((WAVE, SUB, DIM), jnp.float32),
        pltpu.VMEM((WAVE, SUB, DIM), jnp.float32),
        pltpu.VMEM((WAVE, SUB, DIM), jnp.float32),
        pltpu.VMEM((BPW,), jnp.float32),
        pltpu.VMEM((BPW,), jnp.float32),
        pltpu.SemaphoreType.DMA,
    ],
)
def _bpr_sc(bu, bp, bn, ue, ie, pos_out, neg_out,
            idx_u, idx_p, idx_n, u_slab, p_slab, n_slab,
            pos_s, neg_s, sem):
    wid = lax.axis_index("s") * NC + lax.axis_index("c")
    base = wid * BPW

    descs = []
    for src, dst in ((bu, idx_u), (bp, idx_p), (bn, idx_n)):
        for j in range(NCH):
            descs.append(
                pltpu.async_copy(src.at[pl.ds(base + j * CH, CH)],
                                 dst.at[pl.ds(j * CH, CH)], sem))
    for d in descs:
        d.wait()

    lane = lax.iota(jnp.int32, L)

    def wave(w, carry):
        k0 = pl.multiple_of(w * WAVE, WAVE)
        vu = idx_u[pl.ds(k0, WAVE)]
        vp = idx_p[pl.ds(k0, WAVE)]
        vn = idx_n[pl.ds(k0, WAVE)]
        descs = []
        subs = []
        for i in range(WAVE):
            trip = []
            for vec, tab, slab in ((vu, ue, u_slab), (vp, ie, p_slab),
                                   (vn, ie, n_slab)):
                r = vec[i]
                r8 = pl.multiple_of((r >> 3) << 3, SUB)
                trip.append(r & 7)
                descs.append(
                    pltpu.async_copy(tab.at[pl.ds(r8, SUB), :],
                                     slab.at[i], sem))
            subs.append(trip)
        for d in descs:
            d.wait()

        for g in range(WAVE // L):
            pp, pn = [], []
            for li in range(L):
                i = g * L + li
                su, sp, sn = subs[i]
                acc_p = acc_n = None
                for c in range(DIM // L):
                    u = u_slab[i, su, pl.ds(c * L, L)]
                    tp = u * p_slab[i, sp, pl.ds(c * L, L)]
                    tn = u * n_slab[i, sn, pl.ds(c * L, L)]
                    acc_p = tp if acc_p is None else acc_p + tp
                    acc_n = tn if acc_n is None else acc_n + tn
                pp.append(acc_p)
                pn.append(acc_n)
            pos_s[pl.ds(k0 + g * L, L)] = _fold16(pp, lane)
            neg_s[pl.ds(k0 + g * L, L)] = _fold16(pn, lane)
        return carry

    lax.fori_loop(0, BPW // WAVE, wave, 0)

    pltpu.sync_copy(pos_s, pos_out.at[pl.ds(base, BPW)])
    pltpu.sync_copy(neg_s, neg_out.at[pl.ds(base, BPW)])


def _tr_body(x_ref, o_ref):
    o_ref[...] = x_ref[...].T


def _transpose_table(tt):
    n = tt.shape[1]
    blk = 4096
    return pl.pallas_call(
        _tr_body,
        grid=(pl.cdiv(n, blk),),
        in_specs=[pl.BlockSpec((DIM, blk), lambda j: (0, j))],
        out_specs=pl.BlockSpec((blk, DIM), lambda j: (j, 0)),
        out_shape=jax.ShapeDtypeStruct((n, DIM), jnp.float32),
    )(tt)


def kernel(batch_user, batch_pos_item, batch_neg_item, user_emb, item_emb):
    ue = _transpose_table(user_emb.T)
    ie = _transpose_table(item_emb.T)
    pos, neg = _bpr_sc(batch_user, batch_pos_item, batch_neg_item, ue, ie)
    return pos.reshape(BATCH, 1), neg.reshape(BATCH, 1)


# speedup vs baseline: 1.2573x; 1.2292x over previous
import functools

import jax
import jax.numpy as jnp
from jax import lax
from jax.experimental import pallas as pl
from jax.experimental.pallas import tpu as pltpu
from jax.experimental.pallas import tpu_sc as plsc

BATCH = 16384
DIM = 64
L = 16
SUB = 8

_info = plsc.get_sparse_core_info()
NC, NS = _info.num_cores, _info.num_subcores
NW = NC * NS
BPW = BATCH // NW
CH = 128
NCH = BPW // CH
WAVE = 32

_mesh = plsc.VectorSubcoreMesh(core_axis_name="c", subcore_axis_name="s")


_GDN = lax.GatherDimensionNumbers(
    offset_dims=(), collapsed_slice_dims=(0,), start_index_map=(0,))


def _permute(v, idx):
    return lax.gather(v, idx[:, None], dimension_numbers=_GDN,
                      slice_sizes=(1,),
                      mode=lax.GatherScatterMode.PROMISE_IN_BOUNDS)


def _fold16(vecs, lane):
    def rot(v, d):
        return _permute(v, (lane + d) & (L - 1))

    dist = L // 2
    while len(vecs) > 1:
        mask = (lane & (2 * dist - 1)) < dist
        nxt = []
        for i in range(0, len(vecs), 2):
            a2 = vecs[i] + rot(vecs[i], dist)
            b2 = vecs[i + 1] + rot(vecs[i + 1], L - dist)
            nxt.append(jnp.where(mask, a2, b2))
        vecs = nxt
        dist //= 2
    bitrev = (((lane & 1) << 3) | ((lane & 2) << 1)
              | ((lane & 4) >> 1) | ((lane & 8) >> 3))
    return _permute(vecs[0], bitrev)


@functools.partial(
    pl.kernel,
    mesh=_mesh,
    out_type=(
        jax.ShapeDtypeStruct((BATCH,), jnp.float32),
        jax.ShapeDtypeStruct((BATCH,), jnp.float32),
    ),
    scratch_types=[
        pltpu.VMEM((BPW,), jnp.int32),
        pltpu.VMEM((BPW,), jnp.int32),
        pltpu.VMEM((BPW,), jnp.int32),
        pltpu.VMEM((WAVE, SUB, DIM), jnp.float32),
        pltpu.VMEM((WAVE, SUB, DIM), jnp.float32),
        pltpu.VMEM((WAVE, SUB, DIM), jnp.float32),
        pltpu.VMEM((BPW,), jnp.float32),
        pltpu.VMEM((BPW,), jnp.float32),
        pltpu.SemaphoreType.DMA,
    ],
)
def _bpr_sc(bu, bp, bn, ue, ie, pos_out, neg_out,
            idx_u, idx_p, idx_n, u_slab, p_slab, n_slab,
            pos_s, neg_s, sem):
    wid = lax.axis_index("s") * NC + lax.axis_index("c")
    base = wid * BPW

    descs = []
    for src, dst in ((bu, idx_u), (bp, idx_p), (bn, idx_n)):
        for j in range(NCH):
            descs.append(
                pltpu.async_copy(src.at[pl.ds(base + j * CH, CH)],
                                 dst.at[pl.ds(j * CH, CH)], sem))
    for d in descs:
        d.wait()

    lane = lax.iota(jnp.int32, L)

    def wave(w, carry):
        k0 = pl.multiple_of(w * WAVE, WAVE)
        vu = idx_u[pl.ds(k0, WAVE)]
        vp = idx_p[pl.ds(k0, WAVE)]
        vn = idx_n[pl.ds(k0, WAVE)]
        descs = []
        subs = []
        for i in range(WAVE):
            trip = []
            for vec, tab, slab in ((vu, ue, u_slab), (vp, ie, p_slab),
                                   (vn, ie, n_slab)):
                r = vec[i]
                r8 = pl.multiple_of((r >> 3) << 3, SUB)
                trip.append(r & 7)
                descs.append(
                    pltpu.async_copy(tab.at[pl.ds(r8, SUB), :],
                                     slab.at[i], sem))
            subs.append(trip)
        for d in descs:
            d.wait()

        for g in range(WAVE // L):
            pp, pn = [], []
            for li in range(L):
                i = g * L + li
                su, sp, sn = subs[i]
                acc_p = acc_n = None
                for c in range(DIM // L):
                    u = u_slab[i, su, pl.ds(c * L, L)]
                    tp = u * p_slab[i, sp, pl.ds(c * L, L)]
                    tn = u * n_slab[i, sn, pl.ds(c * L, L)]
                    acc_p = tp if acc_p is None else acc_p + tp
                    acc_n = tn if acc_n is None else acc_n + tn
                pp.append(acc_p)
                pn.append(acc_n)
            pos_s[pl.ds(k0 + g * L, L)] = _fold16(pp, lane)
            neg_s[pl.ds(k0 + g * L, L)] = _fold16(pn, lane)
        return carry

    lax.fori_loop(0, BPW // WAVE, wave, 0)

    pltpu.sync_copy(pos_s, pos_out.at[pl.ds(base, BPW)])
    pltpu.sync_copy(neg_s, neg_out.at[pl.ds(base, BPW)])


def _tr_body(x_ref, o_ref):
    o_ref[...] = x_ref[...].T


def _transpose_table(tt):
    n = tt.shape[1]
    blk = 8192
    return pl.pallas_call(
        _tr_body,
        grid=(pl.cdiv(n, blk),),
        in_specs=[pl.BlockSpec((DIM, blk), lambda j: (0, j))],
        out_specs=pl.BlockSpec((blk, DIM), lambda j: (j, 0)),
        out_shape=jax.ShapeDtypeStruct((n, DIM), jnp.float32),
    )(tt)


def kernel(batch_user, batch_pos_item, batch_neg_item, user_emb, item_emb):
    ue = _transpose_table(user_emb.T)
    ie = _transpose_table(item_emb.T)
    pos, neg = _bpr_sc(batch_user, batch_pos_item, batch_neg_item, ue, ie)
    return pos.reshape(BATCH, 1), neg.reshape(BATCH, 1)


# speedup vs baseline: 1.3448x; 1.0696x over previous
import functools

import jax
import jax.numpy as jnp
from jax import lax
from jax.experimental import pallas as pl
from jax.experimental.pallas import tpu as pltpu
from jax.experimental.pallas import tpu_sc as plsc

BATCH = 16384
DIM = 64
L = 16
SUB = 8

_info = plsc.get_sparse_core_info()
NC, NS = _info.num_cores, _info.num_subcores
NW = NC * NS
BPW = BATCH // NW
CH = 128
NCH = BPW // CH
WAVE = 32

_mesh = plsc.VectorSubcoreMesh(core_axis_name="c", subcore_axis_name="s")


_GDN = lax.GatherDimensionNumbers(
    offset_dims=(), collapsed_slice_dims=(0,), start_index_map=(0,))


def _permute(v, idx):
    return lax.gather(v, idx[:, None], dimension_numbers=_GDN,
                      slice_sizes=(1,),
                      mode=lax.GatherScatterMode.PROMISE_IN_BOUNDS)


def _fold16(vecs, lane):
    def rot(v, d):
        return _permute(v, (lane + d) & (L - 1))

    dist = L // 2
    while len(vecs) > 1:
        mask = (lane & (2 * dist - 1)) < dist
        nxt = []
        for i in range(0, len(vecs), 2):
            a2 = vecs[i] + rot(vecs[i], dist)
            b2 = vecs[i + 1] + rot(vecs[i + 1], L - dist)
            nxt.append(jnp.where(mask, a2, b2))
        vecs = nxt
        dist //= 2
    bitrev = (((lane & 1) << 3) | ((lane & 2) << 1)
              | ((lane & 4) >> 1) | ((lane & 8) >> 3))
    return _permute(vecs[0], bitrev)


@functools.partial(
    pl.kernel,
    mesh=_mesh,
    out_type=(
        jax.ShapeDtypeStruct((BATCH,), jnp.float32),
        jax.ShapeDtypeStruct((BATCH,), jnp.float32),
    ),
    scratch_types=[
        pltpu.VMEM((BPW,), jnp.int32),
        pltpu.VMEM((BPW,), jnp.int32),
        pltpu.VMEM((BPW,), jnp.int32),
        pltpu.VMEM((WAVE, SUB, DIM), jnp.float32),
        pltpu.VMEM((WAVE, SUB, DIM), jnp.float32),
        pltpu.VMEM((WAVE, SUB, DIM), jnp.float32),
        pltpu.VMEM((BPW,), jnp.float32),
        pltpu.VMEM((BPW,), jnp.float32),
        pltpu.SemaphoreType.DMA,
    ],
)
def _bpr_sc(bu, bp, bn, ue, ie, pos_out, neg_out,
            idx_u, idx_p, idx_n, u_slab, p_slab, n_slab,
            pos_s, neg_s, sem):
    wid = lax.axis_index("s") * NC + lax.axis_index("c")
    base = wid * BPW

    descs = []
    for src, dst in ((bu, idx_u), (bp, idx_p), (bn, idx_n)):
        for j in range(NCH):
            descs.append(
                pltpu.async_copy(src.at[pl.ds(base + j * CH, CH)],
                                 dst.at[pl.ds(j * CH, CH)], sem))
    for d in descs:
        d.wait()

    lane = lax.iota(jnp.int32, L)

    def wave(w, carry):
        k0 = pl.multiple_of(w * WAVE, WAVE)
        vu = idx_u[pl.ds(k0, WAVE)]
        vp = idx_p[pl.ds(k0, WAVE)]
        vn = idx_n[pl.ds(k0, WAVE)]
        descs = []
        subs = []
        for i in range(WAVE):
            trip = []
            for vec, tab, slab in ((vu, ue, u_slab), (vp, ie, p_slab),
                                   (vn, ie, n_slab)):
                r = vec[i]
                r8 = pl.multiple_of((r >> 3) << 3, SUB)
                trip.append(r & 7)
                descs.append(
                    pltpu.async_copy(tab.at[pl.ds(r8, SUB), :],
                                     slab.at[i], sem))
            subs.append(trip)
        for d in descs:
            d.wait()

        for g in range(WAVE // L):
            pp, pn = [], []
            for li in range(L):
                i = g * L + li
                su, sp, sn = subs[i]
                acc_p = acc_n = None
                for c in range(DIM // L):
                    u = u_slab[i, su, pl.ds(c * L, L)]
                    tp = u * p_slab[i, sp, pl.ds(c * L, L)]
                    tn = u * n_slab[i, sn, pl.ds(c * L, L)]
                    acc_p = tp if acc_p is None else acc_p + tp
                    acc_n = tn if acc_n is None else acc_n + tn
                pp.append(acc_p)
                pn.append(acc_n)
            pos_s[pl.ds(k0 + g * L, L)] = _fold16(pp, lane)
            neg_s[pl.ds(k0 + g * L, L)] = _fold16(pn, lane)
        return carry

    lax.fori_loop(0, BPW // WAVE, wave, 0)

    pltpu.sync_copy(pos_s, pos_out.at[pl.ds(base, BPW)])
    pltpu.sync_copy(neg_s, neg_out.at[pl.ds(base, BPW)])


def _tr_body(x_ref, o_ref):
    o_ref[...] = x_ref[...].T


def _transpose_table(tt):
    n = tt.shape[1]
    blk = 16384
    return pl.pallas_call(
        _tr_body,
        grid=(pl.cdiv(n, blk),),
        in_specs=[pl.BlockSpec((DIM, blk), lambda j: (0, j))],
        out_specs=pl.BlockSpec((blk, DIM), lambda j: (j, 0)),
        out_shape=jax.ShapeDtypeStruct((n, DIM), jnp.float32),
    )(tt)


def kernel(batch_user, batch_pos_item, batch_neg_item, user_emb, item_emb):
    ue = _transpose_table(user_emb.T)
    ie = _transpose_table(item_emb.T)
    pos, neg = _bpr_sc(batch_user, batch_pos_item, batch_neg_item, ue, ie)
    return pos.reshape(BATCH, 1), neg.reshape(BATCH, 1)


# speedup vs baseline: 1.3706x; 1.0191x over previous
import functools

import jax
import jax.numpy as jnp
from jax import lax
from jax.experimental import pallas as pl
from jax.experimental.pallas import tpu as pltpu
from jax.experimental.pallas import tpu_sc as plsc

BATCH = 16384
DIM = 64
L = 16
SUB = 8

_info = plsc.get_sparse_core_info()
NC, NS = _info.num_cores, _info.num_subcores
NW = NC * NS
BPW = BATCH // NW
CH = 128
NCH = BPW // CH
WAVE = 32

_mesh = plsc.VectorSubcoreMesh(core_axis_name="c", subcore_axis_name="s")


_GDN = lax.GatherDimensionNumbers(
    offset_dims=(), collapsed_slice_dims=(0,), start_index_map=(0,))


def _permute(v, idx):
    return lax.gather(v, idx[:, None], dimension_numbers=_GDN,
                      slice_sizes=(1,),
                      mode=lax.GatherScatterMode.PROMISE_IN_BOUNDS)


def _fold16(vecs, lane):
    def rot(v, d):
        return _permute(v, (lane + d) & (L - 1))

    dist = L // 2
    while len(vecs) > 1:
        mask = (lane & (2 * dist - 1)) < dist
        nxt = []
        for i in range(0, len(vecs), 2):
            a2 = vecs[i] + rot(vecs[i], dist)
            b2 = vecs[i + 1] + rot(vecs[i + 1], L - dist)
            nxt.append(jnp.where(mask, a2, b2))
        vecs = nxt
        dist //= 2
    bitrev = (((lane & 1) << 3) | ((lane & 2) << 1)
              | ((lane & 4) >> 1) | ((lane & 8) >> 3))
    return _permute(vecs[0], bitrev)


@functools.partial(
    pl.kernel,
    mesh=_mesh,
    out_type=(
        jax.ShapeDtypeStruct((BATCH,), jnp.float32),
        jax.ShapeDtypeStruct((BATCH,), jnp.float32),
    ),
    scratch_types=[
        pltpu.VMEM((BPW,), jnp.int32),
        pltpu.VMEM((BPW,), jnp.int32),
        pltpu.VMEM((BPW,), jnp.int32),
        pltpu.VMEM((WAVE, SUB, DIM), jnp.float32),
        pltpu.VMEM((WAVE, SUB, DIM), jnp.float32),
        pltpu.VMEM((WAVE, SUB, DIM), jnp.float32),
        pltpu.VMEM((BPW,), jnp.float32),
        pltpu.VMEM((BPW,), jnp.float32),
        pltpu.SemaphoreType.DMA,
    ],
)
def _bpr_sc(bu, bp, bn, ue, ie, pos_out, neg_out,
            idx_u, idx_p, idx_n, u_slab, p_slab, n_slab,
            pos_s, neg_s, sem):
    wid = lax.axis_index("s") * NC + lax.axis_index("c")
    base = wid * BPW

    descs = []
    for src, dst in ((bu, idx_u), (bp, idx_p), (bn, idx_n)):
        for j in range(NCH):
            descs.append(
                pltpu.async_copy(src.at[pl.ds(base + j * CH, CH)],
                                 dst.at[pl.ds(j * CH, CH)], sem))
    for d in descs:
        d.wait()

    lane = lax.iota(jnp.int32, L)

    def wave(w, carry):
        k0 = pl.multiple_of(w * WAVE, WAVE)
        vu = idx_u[pl.ds(k0, WAVE)]
        vp = idx_p[pl.ds(k0, WAVE)]
        vn = idx_n[pl.ds(k0, WAVE)]
        descs = []
        subs = []
        for i in range(WAVE):
            trip = []
            for vec, tab, slab in ((vu, ue, u_slab), (vp, ie, p_slab),
                                   (vn, ie, n_slab)):
                r = vec[i]
                r8 = pl.multiple_of((r >> 3) << 3, SUB)
                trip.append(r & 7)
                descs.append(
                    pltpu.async_copy(tab.at[pl.ds(r8, SUB), :],
                                     slab.at[i], sem))
            subs.append(trip)
        for d in descs:
            d.wait()

        for g in range(WAVE // L):
            pp, pn = [], []
            for li in range(L):
                i = g * L + li
                su, sp, sn = subs[i]
                acc_p = acc_n = None
                for c in range(DIM // L):
                    u = u_slab[i, su, pl.ds(c * L, L)]
                    tp = u * p_slab[i, sp, pl.ds(c * L, L)]
                    tn = u * n_slab[i, sn, pl.ds(c * L, L)]
                    acc_p = tp if acc_p is None else acc_p + tp
                    acc_n = tn if acc_n is None else acc_n + tn
                pp.append(acc_p)
                pn.append(acc_n)
            pos_s[pl.ds(k0 + g * L, L)] = _fold16(pp, lane)
            neg_s[pl.ds(k0 + g * L, L)] = _fold16(pn, lane)
        return carry

    lax.fori_loop(0, BPW // WAVE, wave, 0)

    pltpu.sync_copy(pos_s, pos_out.at[pl.ds(base, BPW)])
    pltpu.sync_copy(neg_s, neg_out.at[pl.ds(base, BPW)])


def _tr_body(x_ref, o_ref):
    o_ref[...] = x_ref[...].T


def _transpose_table(tt):
    n = tt.shape[1]
    blk = 32768
    return pl.pallas_call(
        _tr_body,
        grid=(pl.cdiv(n, blk),),
        in_specs=[pl.BlockSpec((DIM, blk), lambda j: (0, j))],
        out_specs=pl.BlockSpec((blk, DIM), lambda j: (j, 0)),
        out_shape=jax.ShapeDtypeStruct((n, DIM), jnp.float32),
    )(tt)


def kernel(batch_user, batch_pos_item, batch_neg_item, user_emb, item_emb):
    ue = _transpose_table(user_emb.T)
    ie = _transpose_table(item_emb.T)
    pos, neg = _bpr_sc(batch_user, batch_pos_item, batch_neg_item, ue, ie)
    return pos.reshape(BATCH, 1), neg.reshape(BATCH, 1)


# speedup vs baseline: 1.4156x; 1.0328x over previous
import functools

import jax
import jax.numpy as jnp
from jax import lax
from jax.experimental import pallas as pl
from jax.experimental.pallas import tpu as pltpu
from jax.experimental.pallas import tpu_sc as plsc

BATCH = 16384
DIM = 64
L = 16
SUB = 8

_info = plsc.get_sparse_core_info()
NC, NS = _info.num_cores, _info.num_subcores
NW = NC * NS
BPW = BATCH // NW
CH = 128
NCH = BPW // CH
WAVE = 16
NWAVES = BPW // WAVE

_mesh = plsc.VectorSubcoreMesh(core_axis_name="c", subcore_axis_name="s")


_GDN = lax.GatherDimensionNumbers(
    offset_dims=(), collapsed_slice_dims=(0,), start_index_map=(0,))


def _permute(v, idx):
    return lax.gather(v, idx[:, None], dimension_numbers=_GDN,
                      slice_sizes=(1,),
                      mode=lax.GatherScatterMode.PROMISE_IN_BOUNDS)


def _fold16(vecs, lane):
    def rot(v, d):
        return _permute(v, (lane + d) & (L - 1))

    dist = L // 2
    while len(vecs) > 1:
        mask = (lane & (2 * dist - 1)) < dist
        nxt = []
        for i in range(0, len(vecs), 2):
            a2 = vecs[i] + rot(vecs[i], dist)
            b2 = vecs[i + 1] + rot(vecs[i + 1], L - dist)
            nxt.append(jnp.where(mask, a2, b2))
        vecs = nxt
        dist //= 2
    bitrev = (((lane & 1) << 3) | ((lane & 2) << 1)
              | ((lane & 4) >> 1) | ((lane & 8) >> 3))
    return _permute(vecs[0], bitrev)


@functools.partial(
    pl.kernel,
    mesh=_mesh,
    out_type=(
        jax.ShapeDtypeStruct((BATCH,), jnp.float32),
        jax.ShapeDtypeStruct((BATCH,), jnp.float32),
    ),
    scratch_types=[
        pltpu.VMEM((BPW,), jnp.int32),
        pltpu.VMEM((BPW,), jnp.int32),
        pltpu.VMEM((BPW,), jnp.int32),
        pltpu.VMEM((2, WAVE, SUB, DIM), jnp.float32),
        pltpu.VMEM((2, WAVE, SUB, DIM), jnp.float32),
        pltpu.VMEM((2, WAVE, SUB, DIM), jnp.float32),
        pltpu.VMEM((BPW,), jnp.float32),
        pltpu.VMEM((BPW,), jnp.float32),
        pltpu.SemaphoreType.DMA,
    ],
)
def _bpr_sc(bu, bp, bn, ue, ie, pos_out, neg_out,
            idx_u, idx_p, idx_n, u_slab, p_slab, n_slab,
            pos_s, neg_s, sem):
    wid = lax.axis_index("s") * NC + lax.axis_index("c")
    base = wid * BPW

    descs = []
    for src, dst in ((bu, idx_u), (bp, idx_p), (bn, idx_n)):
        for j in range(NCH):
            descs.append(
                pltpu.async_copy(src.at[pl.ds(base + j * CH, CH)],
                                 dst.at[pl.ds(j * CH, CH)], sem))
    for d in descs:
        d.wait()

    lane = lax.iota(jnp.int32, L)

    def fire(w, buf):
        k0 = pl.multiple_of(w * WAVE, WAVE)
        subs = []
        for src, tab, slab in ((idx_u, ue, u_slab), (idx_p, ie, p_slab),
                               (idx_n, ie, n_slab)):
            vec = src[pl.ds(k0, WAVE)]
            for i in range(WAVE):
                r = vec[i]
                r8 = pl.multiple_of((r >> 3) << 3, SUB)
                pltpu.async_copy(tab.at[pl.ds(r8, SUB), :],
                                 slab.at[buf, i], sem)
            subs.append(vec & 7)
        return subs

    def drain(buf):
        for slab, tab in ((u_slab, ue), (p_slab, ie), (n_slab, ie)):
            for i in range(WAVE):
                pltpu.make_async_copy(tab.at[pl.ds(0, SUB), :],
                                      slab.at[buf, i], sem).wait()

    fire(0, 0)

    def wave(w, carry):
        buf = w & 1

        @pl.when(w + 1 < NWAVES)
        def _():
            fire(w + 1, (w + 1) & 1)

        k0 = pl.multiple_of(w * WAVE, WAVE)
        vu = idx_u[pl.ds(k0, WAVE)]
        vp = idx_p[pl.ds(k0, WAVE)]
        vn = idx_n[pl.ds(k0, WAVE)]
        drain(buf)

        pp, pn = [], []
        for i in range(WAVE):
            su = vu[i] & 7
            sp = vp[i] & 7
            sn = vn[i] & 7
            acc_p = acc_n = None
            for c in range(DIM // L):
                u = u_slab[buf, i, su, pl.ds(c * L, L)]
                tp = u * p_slab[buf, i, sp, pl.ds(c * L, L)]
                tn = u * n_slab[buf, i, sn, pl.ds(c * L, L)]
                acc_p = tp if acc_p is None else acc_p + tp
                acc_n = tn if acc_n is None else acc_n + tn
            pp.append(acc_p)
            pn.append(acc_n)
        pos_s[pl.ds(k0, WAVE)] = _fold16(pp, lane)
        neg_s[pl.ds(k0, WAVE)] = _fold16(pn, lane)
        return carry

    lax.fori_loop(0, NWAVES, wave, 0)

    pltpu.sync_copy(pos_s, pos_out.at[pl.ds(base, BPW)])
    pltpu.sync_copy(neg_s, neg_out.at[pl.ds(base, BPW)])


def _tr_body(x_ref, o_ref):
    o_ref[...] = x_ref[...].T


def _transpose_table(tt):
    n = tt.shape[1]
    blk = 32768
    return pl.pallas_call(
        _tr_body,
        grid=(pl.cdiv(n, blk),),
        in_specs=[pl.BlockSpec((DIM, blk), lambda j: (0, j))],
        out_specs=pl.BlockSpec((blk, DIM), lambda j: (j, 0)),
        out_shape=jax.ShapeDtypeStruct((n, DIM), jnp.float32),
    )(tt)


def kernel(batch_user, batch_pos_item, batch_neg_item, user_emb, item_emb):
    ue = _transpose_table(user_emb.T)
    ie = _transpose_table(item_emb.T)
    pos, neg = _bpr_sc(batch_user, batch_pos_item, batch_neg_item, ue, ie)
    return pos.reshape(BATCH, 1), neg.reshape(BATCH, 1)
